# baseline (speedup 1.0000x reference)
import functools

import jax
import jax.numpy as jnp
from jax import lax
from jax.experimental import pallas as pl
from jax.experimental.pallas import tpu as pltpu, tpu_sc as plsc

_V = 8192
_D = 64
_B = 4096
_S = 200
_SO = _S // 8
_BO = _B // 128
_NSQ = 4
_SPW = _S // _NSQ
_MULT = 1315423911


def _body(xt_hbm, tab_hbm, g_hbm, out_hbm,
          slab_v, xrow_v, out_v, g_v, sem_o):
    c = lax.axis_index("c")
    s_ax = lax.axis_index("s")
    wid = s_ax * 2 + c
    lane = lax.iota(jnp.int32, 16)
    i32 = jnp.int32

    eo = wid >> i32(2)
    squad = wid & i32(3)
    s0 = squad * i32(_SPW)

    pltpu.sync_copy(tab_hbm.at[eo], slab_v)

    pltpu.sync_copy(g_hbm, g_v)
    sgl = []
    for ei in range(8):
        gsplat = plsc.load_gather(g_v, [eo * 8 + i32(ei) + 0 * lane])
        sgl.append(1.0 / (1.0 + jnp.exp(-gsplat)))

    def _scale(vo, _):
        for ei in range(8):
            for q in range(8):
                off = ei * 128 + q * 16
                slab_v[vo, pl.ds(off, 16)] = slab_v[vo, pl.ds(off, 16)] * sgl[ei]
        return 0

    lax.fori_loop(i32(0), i32(_D), _scale, 0)

    @pl.when(s0 > i32(0))
    def _prime_prev():
        sp = s0 - i32(1)
        pltpu.sync_copy(
            xt_hbm.at[sp >> i32(3), pl.ds(0, _BO),
                      pl.ds((sp & i32(7)) * i32(128), 128)],
            xrow_v.at[i32(0)])

    def _srow(t, _):
        s = s0 + t
        cp = (t + i32(1)) & i32(1)
        pp = t & i32(1)
        pltpu.sync_copy(
            xt_hbm.at[s >> i32(3), pl.ds(0, _BO),
                      pl.ds((s & i32(7)) * i32(128), 128)],
            xrow_v.at[cp])
        m = jnp.where(s > 0, i32(_MULT), i32(0)) + 0 * lane

        for half in range(2):
            @pl.when(t >= i32(1))
            def _drain(half=half):
                pltpu.make_async_copy(
                    out_v.at[i32(half)],
                    out_hbm.at[i32(0), i32(0), pl.ds(0, _BO // 2)],
                    sem_o.at[i32(half)]).wait()

            def _grp(g4, _, half=half):
                for u in range(4):
                    grp = g4 * i32(4) + i32(u)
                    bo16 = grp >> i32(3)
                    bo = i32(half * 16) + bo16
                    boff = (grp & i32(7)) * i32(16)
                    cur = xrow_v[cp, bo, pl.ds(boff, 16)]
                    prev = xrow_v[pp, bo, pl.ds(boff, 16)]
                    h = (prev * m + cur) & i32(_V - 1)
                    vo = h >> i32(7)
                    col = h & i32(127)
                    for ei in range(8):
                        val = plsc.load_gather(slab_v, [vo, col + i32(ei * 128)])
                        out_v[i32(half), bo16,
                              pl.ds(i32(ei * 128) + boff, 16)] = val
                return 0

            lax.fori_loop(i32(0), i32(32), _grp, 0)

            pltpu.async_copy(
                out_v.at[i32(half)],
                out_hbm.at[s, eo, pl.ds(i32(half * 16), _BO // 2)],
                sem_o.at[i32(half)])
        return 0

    lax.fori_loop(i32(0), i32(_SPW), _srow, 0)

    for half in range(2):
        pltpu.make_async_copy(
            out_v.at[jnp.int32(half)],
            out_hbm.at[jnp.int32(0), jnp.int32(0), pl.ds(0, _BO // 2)],
            sem_o.at[jnp.int32(half)]).wait()


_call = pl.kernel(
    _body,
    out_type=jax.ShapeDtypeStruct((_S, 8, _BO, 1024), jnp.float32),
    mesh=plsc.VectorSubcoreMesh(core_axis_name="c", subcore_axis_name="s"),
    scratch_types=[
        pltpu.VMEM((_D, 1024), jnp.float32),
        pltpu.VMEM((2, _BO, 128), jnp.int32),
        pltpu.VMEM((2, _BO // 2, 1024), jnp.float32),
        pltpu.VMEM((_D,), jnp.float32),
        pltpu.SemaphoreType.DMA((2,)),
    ],
    compiler_params=pltpu.CompilerParams(use_tc_tiling_on_sc=False,
                                         needs_layout_passes=False),
)


@jax.jit
def kernel(x, emb, g):
    xt = (x.T.astype(jnp.int32)
          .reshape(_SO, 8, _BO, 128).transpose(0, 2, 1, 3)
          .reshape(_SO, _BO, 1024))
    tabt = (emb.astype(jnp.float32).T
            .reshape(8, 8, _V // 128, 128).transpose(0, 2, 1, 3)
            .reshape(8, _D, 1024))
    out = _call(xt, tabt, g.astype(jnp.float32))
    return (out.reshape(_S, 8, _BO, 8, 128)
            .transpose(2, 4, 0, 1, 3).reshape(_B, _S, _D))


# speedup vs baseline: 3.2951x; 3.2951x over previous
import functools

import jax
import jax.numpy as jnp
from jax import lax
from jax.experimental import pallas as pl
from jax.experimental.pallas import tpu as pltpu, tpu_sc as plsc

_V = 8192
_D = 64
_B = 4096
_S = 200
_SO = _S // 8
_BO = _B // 128
_NSQ = 4
_SPW = _S // _NSQ
_MULT = 1315423911


def _body(xt_hbm, tab_hbm, g_hbm, out_hbm,
          slab_v, xrow_v, out_v, g_v, sem_o):
    c = lax.axis_index("c")
    s_ax = lax.axis_index("s")
    wid = s_ax * 2 + c
    lane = lax.iota(jnp.int32, 16)
    i32 = jnp.int32

    eo = wid >> i32(2)
    squad = wid & i32(3)
    s0 = squad * i32(_SPW)

    pltpu.sync_copy(tab_hbm.at[eo], slab_v)

    pltpu.sync_copy(g_hbm, g_v)
    sgl = []
    for ei in range(8):
        gsplat = plsc.load_gather(g_v, [eo * 8 + i32(ei) + 0 * lane])
        sgl.append(1.0 / (1.0 + jnp.exp(-gsplat)))

    @plsc.parallel_loop(jnp.int32(0), jnp.int32(_D), jnp.int32(1), unroll=2)
    def _scale(vo):
        base = vo * i32(1024)
        for ei in range(8):
            for q in range(8):
                off = base + i32(ei * 128 + q * 16)
                slab_v[pl.ds(off, 16)] = slab_v[pl.ds(off, 16)] * sgl[ei]

    @pl.when(s0 > i32(0))
    def _prime_prev():
        sp = s0 - i32(1)
        pltpu.sync_copy(
            xt_hbm.at[sp >> i32(3), pl.ds(0, _BO),
                      pl.ds((sp & i32(7)) * i32(128), 128)],
            xrow_v.at[i32(0)])

    def _srow(t, _):
        s = s0 + t
        cp = (t + i32(1)) & i32(1)
        pp = t & i32(1)
        pltpu.sync_copy(
            xt_hbm.at[s >> i32(3), pl.ds(0, _BO),
                      pl.ds((s & i32(7)) * i32(128), 128)],
            xrow_v.at[cp])
        m = jnp.where(s > 0, i32(_MULT), i32(0)) + 0 * lane

        for half in range(2):
            @pl.when(t >= i32(1))
            def _drain(half=half):
                pltpu.make_async_copy(
                    out_v.at[i32(half)],
                    out_hbm.at[i32(0), i32(0), pl.ds(0, _BO // 2)],
                    sem_o.at[i32(half)]).wait()

            @plsc.parallel_loop(jnp.int32(0), jnp.int32(128), jnp.int32(1), unroll=4)
            def _grp(grp, half=half):
                bo16 = grp >> i32(3)
                bo = i32(half * 16) + bo16
                boff = (grp & i32(7)) * i32(16)
                cur = xrow_v[cp, bo, pl.ds(boff, 16)]
                prev = xrow_v[pp, bo, pl.ds(boff, 16)]
                h = (prev * m + cur) & i32(_V - 1)
                base = ((h >> i32(7)) << i32(10)) | (h & i32(127))
                for ei in range(8):
                    val = plsc.load_gather(slab_v, [base + i32(ei * 128)])
                    out_v[i32(half), bo16,
                          pl.ds(i32(ei * 128) + boff, 16)] = val

            pltpu.async_copy(
                out_v.at[i32(half)],
                out_hbm.at[s, eo, pl.ds(i32(half * 16), _BO // 2)],
                sem_o.at[i32(half)])
        return 0

    lax.fori_loop(i32(0), i32(_SPW), _srow, 0)

    for half in range(2):
        pltpu.make_async_copy(
            out_v.at[jnp.int32(half)],
            out_hbm.at[jnp.int32(0), jnp.int32(0), pl.ds(0, _BO // 2)],
            sem_o.at[jnp.int32(half)]).wait()


_call = pl.kernel(
    _body,
    out_type=jax.ShapeDtypeStruct((_S, 8, _BO, 1024), jnp.float32),
    mesh=plsc.VectorSubcoreMesh(core_axis_name="c", subcore_axis_name="s"),
    scratch_types=[
        pltpu.VMEM((_D * 1024,), jnp.float32),
        pltpu.VMEM((2, _BO, 128), jnp.int32),
        pltpu.VMEM((2, _BO // 2, 1024), jnp.float32),
        pltpu.VMEM((_D,), jnp.float32),
        pltpu.SemaphoreType.DMA((2,)),
    ],
    compiler_params=pltpu.CompilerParams(use_tc_tiling_on_sc=False,
                                         needs_layout_passes=False),
)


@jax.jit
def kernel(x, emb, g):
    xt = (x.T.astype(jnp.int32)
          .reshape(_SO, 8, _BO, 128).transpose(0, 2, 1, 3)
          .reshape(_SO, _BO, 1024))
    tabt = (emb.astype(jnp.float32).T
            .reshape(8, 8, _V // 128, 128).transpose(0, 2, 1, 3)
            .reshape(8, _D * 1024))
    out = _call(xt, tabt, g.astype(jnp.float32))
    return (out.reshape(_S, 8, _BO, 8, 128)
            .transpose(2, 4, 0, 1, 3).reshape(_B, _S, _D))


# speedup vs baseline: 4.5430x; 1.3787x over previous
import functools

import jax
import jax.numpy as jnp
from jax import lax
from jax.experimental import pallas as pl
from jax.experimental.pallas import tpu as pltpu, tpu_sc as plsc

_V = 8192
_D = 64
_B = 4096
_S = 200
_SO = _S // 8
_BO = _B // 128
_NSQ = 4
_SPW = _S // _NSQ
_MULT = 1315423911


def _body(xt_hbm, tab_hbm, g_hbm, out_hbm,
          slab_v, xrow_v, out_v, g_v, sem_o, sem_x):
    c = lax.axis_index("c")
    s_ax = lax.axis_index("s")
    wid = s_ax * 2 + c
    lane = lax.iota(jnp.int32, 16)
    i32 = jnp.int32

    eo = wid >> i32(2)
    squad = wid & i32(3)
    s0 = squad * i32(_SPW)

    pltpu.sync_copy(tab_hbm.at[eo], slab_v)

    pltpu.sync_copy(g_hbm, g_v)
    sgl = []
    for ei in range(8):
        gsplat = plsc.load_gather(g_v, [eo * 8 + i32(ei) + 0 * lane])
        sgl.append(1.0 / (1.0 + jnp.exp(-gsplat)))

    @plsc.parallel_loop(jnp.int32(0), jnp.int32(_D), jnp.int32(1), unroll=2)
    def _scale(vo):
        base = vo * i32(1024)
        for ei in range(8):
            for q in range(8):
                off = base + i32(ei * 128 + q * 16)
                slab_v[pl.ds(off, 16)] = slab_v[pl.ds(off, 16)] * sgl[ei]

    @pl.when(s0 > i32(0))
    def _prime_prev():
        sp = s0 - i32(1)
        pltpu.sync_copy(
            xt_hbm.at[sp >> i32(3), pl.ds(0, _BO),
                      pl.ds((sp & i32(7)) * i32(128), 128)],
            xrow_v.at[i32(0)])
    pltpu.async_copy(
        xt_hbm.at[s0 >> i32(3), pl.ds(0, _BO),
                  pl.ds((s0 & i32(7)) * i32(128), 128)],
        xrow_v.at[i32(1)], sem_x)

    def _srow(t, _):
        s = s0 + t
        pp = t % i32(3)
        cp = (t + i32(1)) % i32(3)
        np_ = (t + i32(2)) % i32(3)
        pltpu.make_async_copy(
            xt_hbm.at[i32(0), pl.ds(0, _BO), pl.ds(0, 128)],
            xrow_v.at[cp], sem_x).wait()

        @pl.when(t + i32(1) < i32(_SPW))
        def _prefetch():
            sn = s + i32(1)
            pltpu.async_copy(
                xt_hbm.at[sn >> i32(3), pl.ds(0, _BO),
                          pl.ds((sn & i32(7)) * i32(128), 128)],
                xrow_v.at[np_], sem_x)
        m = jnp.where(s > 0, i32(_MULT), i32(0)) + 0 * lane

        for half in range(2):
            @pl.when(t >= i32(1))
            def _drain(half=half):
                pltpu.make_async_copy(
                    out_v.at[i32(half)],
                    out_hbm.at[i32(0), i32(0), pl.ds(0, _BO // 2)],
                    sem_o.at[i32(half)]).wait()

            @plsc.parallel_loop(jnp.int32(0), jnp.int32(128), jnp.int32(1), unroll=4)
            def _grp(grp, half=half):
                bo16 = grp >> i32(3)
                bo = i32(half * 16) + bo16
                boff = (grp & i32(7)) * i32(16)
                cur = xrow_v[cp, bo, pl.ds(boff, 16)]
                prev = xrow_v[pp, bo, pl.ds(boff, 16)]
                h = (prev * m + cur) & i32(_V - 1)
                base = ((h >> i32(7)) << i32(10)) | (h & i32(127))
                for ei in range(8):
                    val = plsc.load_gather(slab_v, [base + i32(ei * 128)])
                    out_v[i32(half), bo16,
                          pl.ds(i32(ei * 128) + boff, 16)] = val

            pltpu.async_copy(
                out_v.at[i32(half)],
                out_hbm.at[s, eo, pl.ds(i32(half * 16), _BO // 2)],
                sem_o.at[i32(half)])
        return 0

    lax.fori_loop(i32(0), i32(_SPW), _srow, 0)

    for half in range(2):
        pltpu.make_async_copy(
            out_v.at[jnp.int32(half)],
            out_hbm.at[jnp.int32(0), jnp.int32(0), pl.ds(0, _BO // 2)],
            sem_o.at[jnp.int32(half)]).wait()


_call = pl.kernel(
    _body,
    out_type=jax.ShapeDtypeStruct((_S, 8, _BO, 1024), jnp.float32),
    mesh=plsc.VectorSubcoreMesh(core_axis_name="c", subcore_axis_name="s"),
    scratch_types=[
        pltpu.VMEM((_D * 1024,), jnp.float32),
        pltpu.VMEM((3, _BO, 128), jnp.int32),
        pltpu.VMEM((2, _BO // 2, 1024), jnp.float32),
        pltpu.VMEM((_D,), jnp.float32),
        pltpu.SemaphoreType.DMA((2,)),
        pltpu.SemaphoreType.DMA,
    ],
    compiler_params=pltpu.CompilerParams(use_tc_tiling_on_sc=False,
                                         needs_layout_passes=False),
)


@jax.jit
def kernel(x, emb, g):
    xt = (x.T.astype(jnp.int32)
          .reshape(_SO, 8, _BO, 128).transpose(0, 2, 1, 3)
          .reshape(_SO, _BO, 1024))
    tabt = (emb.astype(jnp.float32).T
            .reshape(8, 8, _V // 128, 128).transpose(0, 2, 1, 3)
            .reshape(8, _D * 1024))
    out = _call(xt, tabt, g.astype(jnp.float32))
    return (out.reshape(_S, 8, _BO, 8, 128)
            .transpose(2, 4, 0, 1, 3).reshape(_B, _S, _D))


# speedup vs baseline: 4.5839x; 1.0090x over previous
import functools

import jax
import jax.numpy as jnp
from jax import lax
from jax.experimental import pallas as pl
from jax.experimental.pallas import tpu as pltpu, tpu_sc as plsc

_V = 8192
_D = 64
_B = 4096
_S = 200
_SO = _S // 8
_BO = _B // 128
_NSQ = 4
_SPW = _S // _NSQ
_MULT = 1315423911


def _body(xt_hbm, tab_hbm, g_hbm, out_hbm,
          slab_v, xrow_v, out_v, g_v, sem_o, sem_x):
    c = lax.axis_index("c")
    s_ax = lax.axis_index("s")
    wid = s_ax * 2 + c
    lane = lax.iota(jnp.int32, 16)
    i32 = jnp.int32

    eo = wid >> i32(2)
    squad = wid & i32(3)
    s0 = squad * i32(_SPW)

    pltpu.sync_copy(tab_hbm.at[eo], slab_v)

    pltpu.sync_copy(g_hbm, g_v)
    sgl = []
    for ei in range(8):
        gsplat = plsc.load_gather(g_v, [eo * 8 + i32(ei) + 0 * lane])
        sgl.append(1.0 / (1.0 + jnp.exp(-gsplat)))

    @plsc.parallel_loop(jnp.int32(0), jnp.int32(_D), jnp.int32(1), unroll=2)
    def _scale(vo):
        base = vo * i32(1024)
        for ei in range(8):
            for q in range(8):
                off = base + i32(ei * 128 + q * 16)
                slab_v[pl.ds(off, 16)] = slab_v[pl.ds(off, 16)] * sgl[ei]

    @pl.when(s0 > i32(0))
    def _prime_prev():
        sp = s0 - i32(1)
        pltpu.sync_copy(
            xt_hbm.at[sp >> i32(3), pl.ds(0, _BO),
                      pl.ds((sp & i32(7)) * i32(128), 128)],
            xrow_v.at[i32(0)])
    pltpu.async_copy(
        xt_hbm.at[s0 >> i32(3), pl.ds(0, _BO),
                  pl.ds((s0 & i32(7)) * i32(128), 128)],
        xrow_v.at[i32(1)], sem_x)

    def _srow(t, _):
        s = s0 + t
        pp = t % i32(3)
        cp = (t + i32(1)) % i32(3)
        np_ = (t + i32(2)) % i32(3)
        pltpu.make_async_copy(
            xt_hbm.at[i32(0), pl.ds(0, _BO), pl.ds(0, 128)],
            xrow_v.at[cp], sem_x).wait()

        @pl.when(t + i32(1) < i32(_SPW))
        def _prefetch():
            sn = s + i32(1)
            pltpu.async_copy(
                xt_hbm.at[sn >> i32(3), pl.ds(0, _BO),
                          pl.ds((sn & i32(7)) * i32(128), 128)],
                xrow_v.at[np_], sem_x)
        m = jnp.where(s > 0, i32(_MULT), i32(0)) + 0 * lane

        for half in range(2):
            @pl.when(t >= i32(1))
            def _drain(half=half):
                pltpu.make_async_copy(
                    out_v.at[i32(half)],
                    out_hbm.at[i32(0), i32(0), pl.ds(0, _BO // 2)],
                    sem_o.at[i32(half)]).wait()

            @plsc.parallel_loop(jnp.int32(0), jnp.int32(128), jnp.int32(1), unroll=8)
            def _grp(grp, half=half):
                bo16 = grp >> i32(3)
                bo = i32(half * 16) + bo16
                boff = (grp & i32(7)) * i32(16)
                cur = xrow_v[cp, bo, pl.ds(boff, 16)]
                prev = xrow_v[pp, bo, pl.ds(boff, 16)]
                h = (prev * m + cur) & i32(_V - 1)
                base = ((h >> i32(7)) << i32(10)) | (h & i32(127))
                for ei in range(8):
                    val = plsc.load_gather(slab_v, [base + i32(ei * 128)])
                    out_v[i32(half), bo16,
                          pl.ds(i32(ei * 128) + boff, 16)] = val

            pltpu.async_copy(
                out_v.at[i32(half)],
                out_hbm.at[s, eo, pl.ds(i32(half * 16), _BO // 2)],
                sem_o.at[i32(half)])
        return 0

    lax.fori_loop(i32(0), i32(_SPW), _srow, 0)

    for half in range(2):
        pltpu.make_async_copy(
            out_v.at[jnp.int32(half)],
            out_hbm.at[jnp.int32(0), jnp.int32(0), pl.ds(0, _BO // 2)],
            sem_o.at[jnp.int32(half)]).wait()


_call = pl.kernel(
    _body,
    out_type=jax.ShapeDtypeStruct((_S, 8, _BO, 1024), jnp.float32),
    mesh=plsc.VectorSubcoreMesh(core_axis_name="c", subcore_axis_name="s"),
    scratch_types=[
        pltpu.VMEM((_D * 1024,), jnp.float32),
        pltpu.VMEM((3, _BO, 128), jnp.int32),
        pltpu.VMEM((2, _BO // 2, 1024), jnp.float32),
        pltpu.VMEM((_D,), jnp.float32),
        pltpu.SemaphoreType.DMA((2,)),
        pltpu.SemaphoreType.DMA,
    ],
    compiler_params=pltpu.CompilerParams(use_tc_tiling_on_sc=False,
                                         needs_layout_passes=False),
)


@jax.jit
def kernel(x, emb, g):
    xt = (x.T.astype(jnp.int32)
          .reshape(_SO, 8, _BO, 128).transpose(0, 2, 1, 3)
          .reshape(_SO, _BO, 1024))
    tabt = (emb.astype(jnp.float32).T
            .reshape(8, 8, _V // 128, 128).transpose(0, 2, 1, 3)
            .reshape(8, _D * 1024))
    out = _call(xt, tabt, g.astype(jnp.float32))
    return (out.reshape(_S, 8, _BO, 8, 128)
            .transpose(2, 4, 0, 1, 3).reshape(_B, _S, _D))
